# Initial kernel scaffold; baseline (speedup 1.0000x reference)
import jax
import jax.numpy as jnp
from jax.experimental import pallas as pl


def kernel(x, edge_index, batch, Wc0, bc0, g0, be0, Wc1, bc1, g1, be1, Ws1, bs1, Wd0, bd0, gd0, bed0, Wd1, bd1, gd1, bed1, Wd2, bd2, Wp0, bp0, Wp1, bp1):
    raise NotImplementedError("write your pallas kernel here")


# speedup vs baseline: 13.9198x; 13.9198x over previous
import functools

import jax
import jax.numpy as jnp
from jax import lax
from jax.experimental import pallas as pl
from jax.experimental.pallas import tpu as pltpu
from jax.experimental.pallas import tpu_sc as plsc

N = 10000
E = 320000
D = 128
H0 = 128
H1 = 64
G = 64

NC = 2
NS = 16
NW = NC * NS
NPAD = 10240
NPS = NPAD // NS
EPW = E // NW
CH = 128
NFULL = EPW // CH
TAIL = EPW - NFULL * CH


def _zero_vmem(ref, rows, width):
  zv = jnp.zeros((16,), jnp.float32)

  def body(i, _):
    for j in range(width // 16):
      ref[i, pl.ds(j * 16, 16)] = zv
    return 0

  lax.fori_loop(0, rows, body, 0)


def _fill_vmem_1d(ref, rows, val):
  v = jnp.full((16,), val, jnp.float32)

  def body(i, _):
    ref[pl.ds(i * 16, 16)] = v
    return 0

  lax.fori_loop(0, rows // 16, body, 0)


_MESH = plsc.VectorSubcoreMesh(
    core_axis_name="c", subcore_axis_name="s", num_cores=NC, num_subcores=NS
)


@functools.partial(
    pl.kernel,
    out_type=jax.ShapeDtypeStruct((NC, NPAD), jnp.float32),
    mesh=_MESH,
    scratch_types=[
        pltpu.VMEM((CH,), jnp.int32),
        pltpu.VMEM((TAIL,), jnp.int32),
        pltpu.VMEM((CH,), jnp.float32),
        pltpu.VMEM((TAIL,), jnp.float32),
        pltpu.VMEM_SHARED((NPAD,), jnp.float32),
        pltpu.SemaphoreType.DMA,
    ],
)
def _deg_kernel(dst_hbm, out_hbm, didx, didx_t, ones, ones_t, acc, sem):
  c = lax.axis_index("c")
  s = lax.axis_index("s")
  _fill_vmem_1d(ones, CH, 0.0)
  for k in range(NPS // CH):
    pltpu.sync_copy(ones, acc.at[pl.ds(s * NPS + k * CH, CH)])
  plsc.subcore_barrier()
  _fill_vmem_1d(ones, CH, 1.0)
  _fill_vmem_1d(ones_t, TAIL, 1.0)

  ebase = (c * NS + s) * EPW

  def body(j, _):
    pltpu.sync_copy(dst_hbm.at[pl.ds(ebase + j * CH, CH)], didx)
    pltpu.sync_copy(ones, acc.at[didx], add=True)
    return 0

  lax.fori_loop(0, NFULL, body, 0)
  pltpu.sync_copy(dst_hbm.at[pl.ds(ebase + NFULL * CH, TAIL)], didx_t)
  pltpu.sync_copy(ones_t, acc.at[didx_t], add=True)
  plsc.subcore_barrier()
  pltpu.sync_copy(
      acc.at[pl.ds(s * NPS, NPS)], out_hbm.at[c].at[pl.ds(s * NPS, NPS)]
  )


def _make_agg(F):
  params = None if F == 128 else pltpu.CompilerParams(use_tc_tiling_on_sc=False)

  @functools.partial(
      pl.kernel,
      out_type=jax.ShapeDtypeStruct((NC, NPAD, F), jnp.float32),
      mesh=_MESH,
      compiler_params=params,
      scratch_types=[
          pltpu.VMEM((CH,), jnp.int32),
          pltpu.VMEM((CH,), jnp.int32),
          pltpu.VMEM((TAIL,), jnp.int32),
          pltpu.VMEM((TAIL,), jnp.int32),
          pltpu.VMEM((CH, F), jnp.float32),
          pltpu.VMEM((TAIL, F), jnp.float32),
          pltpu.VMEM_SHARED((NPAD, F), jnp.float32),
          pltpu.SemaphoreType.DMA,
      ],
      name=f"edge_agg_f{F}",
  )
  def agg(y_hbm, src_hbm, dst_hbm, out_hbm, sidx, didx, sidx_t, didx_t,
          rows, rows_t, acc, sem):
    c = lax.axis_index("c")
    s = lax.axis_index("s")
    _zero_vmem(rows, CH, F)
    for k in range(NPS // CH):
      pltpu.sync_copy(rows, acc.at[pl.ds(s * NPS + k * CH, CH)])
    plsc.subcore_barrier()

    ebase = (c * NS + s) * EPW

    def body(j, _):
      base = ebase + j * CH
      pltpu.sync_copy(src_hbm.at[pl.ds(base, CH)], sidx)
      pltpu.sync_copy(dst_hbm.at[pl.ds(base, CH)], didx)
      pltpu.async_copy(y_hbm.at[sidx], rows, sem).wait()
      pltpu.sync_copy(rows, acc.at[didx], add=True)
      return 0

    lax.fori_loop(0, NFULL, body, 0)
    base = ebase + NFULL * CH
    pltpu.sync_copy(src_hbm.at[pl.ds(base, TAIL)], sidx_t)
    pltpu.sync_copy(dst_hbm.at[pl.ds(base, TAIL)], didx_t)
    pltpu.async_copy(y_hbm.at[sidx_t], rows_t, sem).wait()
    pltpu.sync_copy(rows_t, acc.at[didx_t], add=True)

    plsc.subcore_barrier()
    pltpu.sync_copy(
        acc.at[pl.ds(s * NPS, NPS)], out_hbm.at[c].at[pl.ds(s * NPS, NPS)]
    )

  return agg


_agg128 = _make_agg(H0)
_agg64 = _make_agg(H1)


def _matmul_t(a, w):
  return lax.dot_general(
      a, w, (((1,), (1,)), ((), ())), preferred_element_type=jnp.float32
  )


def _bn(t, g, b, eps=1e-5):
  m = jnp.mean(t, axis=0, keepdims=True)
  v = jnp.mean((t - m) * (t - m), axis=0, keepdims=True)
  return (t - m) * lax.rsqrt(v + eps) * g[None, :] + b[None, :]


def _tc1_body(x_ref, wc0_ref, degp_ref, y0_ref, dinv_ref):
  deg = degp_ref[0, :N] + degp_ref[1, :N] + 1.0
  dinv = lax.rsqrt(deg)[:, None]
  dinv_ref[...] = dinv
  xw = _matmul_t(x_ref[...], wc0_ref[...])
  y0_ref[:N, :] = xw * dinv
  y0_ref[N:, :] = jnp.zeros((NPAD - N, H0), jnp.float32)


_tc1 = pl.pallas_call(
    _tc1_body,
    out_shape=(
        jax.ShapeDtypeStruct((NPAD, H0), jnp.float32),
        jax.ShapeDtypeStruct((N, 1), jnp.float32),
    ),
)


def _tc2_body(aggp_ref, y0_ref, dinv_ref, x_ref, bc0_ref, g0_ref, be0_ref,
              wc1_ref, ws1_ref, bs1_ref, y1_ref, r1_ref):
  dinv = dinv_ref[...]
  agg = aggp_ref[0, :N, :] + aggp_ref[1, :N, :] + y0_ref[:N, :]
  conv0 = dinv * agg + bc0_ref[...][None, :]
  h = jnp.maximum(_bn(conv0, g0_ref[...], be0_ref[...]), 0.0)
  x1 = jnp.maximum(h + x_ref[...], 0.0)
  y1_ref[:N, :] = _matmul_t(x1, wc1_ref[...]) * dinv
  y1_ref[N:, :] = jnp.zeros((NPAD - N, H1), jnp.float32)
  r1_ref[...] = _matmul_t(x1, ws1_ref[...]) + bs1_ref[...][None, :]


_tc2 = pl.pallas_call(
    _tc2_body,
    out_shape=(
        jax.ShapeDtypeStruct((NPAD, H1), jnp.float32),
        jax.ShapeDtypeStruct((N, H1), jnp.float32),
    ),
)


def _tc3_body(aggp_ref, y1_ref, dinv_ref, r1_ref, batch_ref, bc1_ref, g1_ref,
              be1_ref, wd0_ref, bd0_ref, gd0_ref, bed0_ref, wd1_ref, bd1_ref,
              gd1_ref, bed1_ref, wd2_ref, bd2_ref, wp0_ref, bp0_ref, wp1_ref,
              bp1_ref, z_ref, xr_ref, zg_ref):
  dinv = dinv_ref[...]
  agg = aggp_ref[0, :N, :] + aggp_ref[1, :N, :] + y1_ref[:N, :]
  conv1 = dinv * agg + bc1_ref[...][None, :]
  h = jnp.maximum(_bn(conv1, g1_ref[...], be1_ref[...]), 0.0)
  x2 = jnp.maximum(h + r1_ref[...], 0.0)
  nrm = jnp.sqrt(jnp.sum(x2 * x2, axis=1, keepdims=True))
  z = x2 / jnp.maximum(nrm, 1e-12)
  z_ref[...] = z

  t = jnp.maximum(_matmul_t(z, wd0_ref[...]) + bd0_ref[...][None, :], 0.0)
  t = _bn(t, gd0_ref[...], bed0_ref[...])
  t = jnp.maximum(_matmul_t(t, wd1_ref[...]) + bd1_ref[...][None, :], 0.0)
  t = _bn(t, gd1_ref[...], bed1_ref[...])
  xr_ref[...] = _matmul_t(t, wd2_ref[...]) + bd2_ref[...][None, :]

  batch = batch_ref[...]
  rows_g = lax.broadcasted_iota(jnp.int32, (G, 1), 0)

  def seg_body(g, zg):
    m = batch == g
    row = jnp.max(jnp.where(m, z, -1e30), axis=0)
    return jnp.where(rows_g == g, row[None, :], zg)

  zg = lax.fori_loop(0, G, seg_body, jnp.full((G, H1), -1e30, jnp.float32))
  t = jnp.maximum(_matmul_t(zg, wp0_ref[...]) + bp0_ref[...][None, :], 0.0)
  zg_ref[...] = _matmul_t(t, wp1_ref[...]) + bp1_ref[...][None, :]


_tc3 = pl.pallas_call(
    _tc3_body,
    out_shape=(
        jax.ShapeDtypeStruct((N, H1), jnp.float32),
        jax.ShapeDtypeStruct((N, D), jnp.float32),
        jax.ShapeDtypeStruct((G, H1), jnp.float32),
    ),
)


def kernel(x, edge_index, batch, Wc0, bc0, g0, be0, Wc1, bc1, g1, be1, Ws1,
           bs1, Wd0, bd0, gd0, bed0, Wd1, bd1, gd1, bed1, Wd2, bd2, Wp0, bp0,
           Wp1, bp1):
  src = edge_index[0]
  dst = edge_index[1]
  degp = _deg_kernel(dst)
  y0, dinv = _tc1(x, Wc0, degp)
  aggp0 = _agg128(y0, src, dst)
  y1, r1 = _tc2(aggp0, y0, dinv, x, bc0, g0, be0, Wc1, Ws1, bs1)
  aggp1 = _agg64(y1, src, dst)
  z, x_recon, z_g_mlp = _tc3(
      aggp1, y1, dinv, r1, batch.reshape(N, 1), bc1, g1, be1, Wd0, bd0, gd0,
      bed0, Wd1, bd1, gd1, bed1, Wd2, bd2, Wp0, bp0, Wp1, bp1
  )
  return (z, x_recon, z_g_mlp)


# speedup vs baseline: 19.5179x; 1.4022x over previous
import functools

import jax
import jax.numpy as jnp
from jax import lax
from jax.experimental import pallas as pl
from jax.experimental.pallas import tpu as pltpu
from jax.experimental.pallas import tpu_sc as plsc

N = 10000
E = 320000
D = 128
H0 = 128
H1 = 64
G = 64

NC = 2
NS = 16
NW = NC * NS
NPAD = 10240
NPS = NPAD // NS
EPW = E // NW
CH = 80
NCH = EPW // CH
NB = 5


def _zero_vmem(ref, rows, width):
  zv = jnp.zeros((16,), jnp.float32)

  def body(i, _):
    for j in range(width // 16):
      ref[i, pl.ds(j * 16, 16)] = zv
    return 0

  lax.fori_loop(0, rows, body, 0)


def _fill_vmem_1d(ref, rows, val):
  v = jnp.full((16,), val, jnp.float32)

  def body(i, _):
    ref[pl.ds(i * 16, 16)] = v
    return 0

  lax.fori_loop(0, rows // 16, body, 0)


_MESH = plsc.VectorSubcoreMesh(
    core_axis_name="c", subcore_axis_name="s", num_cores=NC, num_subcores=NS
)


@functools.partial(
    pl.kernel,
    out_type=jax.ShapeDtypeStruct((NC, NPAD), jnp.float32),
    mesh=_MESH,
    scratch_types=[
        pltpu.VMEM((NCH, CH), jnp.int32),
        pltpu.VMEM((CH,), jnp.float32),
        pltpu.VMEM_SHARED((NPAD,), jnp.float32),
        pltpu.SemaphoreType.DMA,
    ],
)
def _deg_kernel(eidx_hbm, out_hbm, didxv, ones, acc, sem):
  c = lax.axis_index("c")
  s = lax.axis_index("s")
  _fill_vmem_1d(ones, CH, 0.0)
  for k in range(NPS // CH):
    pltpu.sync_copy(ones, acc.at[pl.ds(s * NPS + k * CH, CH)])
  plsc.subcore_barrier()
  _fill_vmem_1d(ones, CH, 1.0)

  w = c * NS + s
  pltpu.sync_copy(eidx_hbm.at[1].at[w], didxv)

  def body(j, _):
    pltpu.sync_copy(ones, acc.at[didxv.at[j]], add=True)
    return 0

  lax.fori_loop(0, NCH, body, 0)
  plsc.subcore_barrier()
  pltpu.sync_copy(
      acc.at[pl.ds(s * NPS, NPS)], out_hbm.at[c].at[pl.ds(s * NPS, NPS)]
  )


F_AGG = 64


def _make_agg():
  F = F_AGG
  params = pltpu.CompilerParams(use_tc_tiling_on_sc=False)

  @functools.partial(
      pl.kernel,
      out_type=jax.ShapeDtypeStruct((NC, NPAD, F), jnp.float32),
      mesh=_MESH,
      compiler_params=params,
      scratch_types=[
          pltpu.VMEM((NCH, CH), jnp.int32),
          pltpu.VMEM((NCH, CH), jnp.int32),
          pltpu.VMEM((NB, CH, F), jnp.float32),
          pltpu.VMEM_SHARED((NPAD, F), jnp.float32),
          pltpu.SemaphoreType.DMA,
          pltpu.SemaphoreType.DMA,
          pltpu.SemaphoreType.DMA,
          pltpu.SemaphoreType.DMA,
          pltpu.SemaphoreType.DMA,
      ],
      name=f"edge_agg_f{F}",
  )
  def agg(y_hbm, eidx_hbm, out_hbm, sidxv, didxv, rows, acc,
          sg0, sg1, sg2, sg3, sg4):
    semg = [sg0, sg1, sg2, sg3, sg4]
    c = lax.axis_index("c")
    s = lax.axis_index("s")
    _zero_vmem(rows.at[0], CH, F)
    for k in range(NPS // CH):
      pltpu.sync_copy(rows.at[0], acc.at[pl.ds(s * NPS + k * CH, CH)])
    plsc.subcore_barrier()

    w = c * NS + s
    pltpu.sync_copy(eidx_hbm.at[0].at[w], sidxv)
    pltpu.sync_copy(eidx_hbm.at[1].at[w], didxv)

    def body(t, _):
      descs = []
      for i in range(NB):
        j = t * NB + i
        descs.append(
            pltpu.async_copy(y_hbm.at[sidxv.at[j]], rows.at[i], semg[i])
        )
      for i in range(NB):
        j = t * NB + i
        descs[i].wait()
        pltpu.sync_copy(rows.at[i], acc.at[didxv.at[j]], add=True)
      return 0

    lax.fori_loop(0, NCH // NB, body, 0)
    plsc.subcore_barrier()
    pltpu.sync_copy(
        acc.at[pl.ds(s * NPS, NPS)], out_hbm.at[c].at[pl.ds(s * NPS, NPS)]
    )

  return agg


_edge_agg = _make_agg()


def _matmul_t(a, w):
  return lax.dot_general(
      a, w, (((1,), (1,)), ((), ())), preferred_element_type=jnp.float32
  )


def _bn(t, g, b, eps=1e-5):
  m = jnp.mean(t, axis=0, keepdims=True)
  v = jnp.mean((t - m) * (t - m), axis=0, keepdims=True)
  return (t - m) * lax.rsqrt(v + eps) * g[None, :] + b[None, :]


def _tc1_body(x_ref, wc0_ref, degp_ref, y0a_ref, y0b_ref, dinv_ref):
  deg = degp_ref[0, :N] + degp_ref[1, :N] + 1.0
  dinv = lax.rsqrt(deg)[:, None]
  dinv_ref[...] = dinv
  y0 = _matmul_t(x_ref[...], wc0_ref[...]) * dinv
  zpad = jnp.zeros((NPAD - N, H1), jnp.float32)
  y0a_ref[:N, :] = y0[:, :H1]
  y0a_ref[N:, :] = zpad
  y0b_ref[:N, :] = y0[:, H1:]
  y0b_ref[N:, :] = zpad


_tc1 = pl.pallas_call(
    _tc1_body,
    out_shape=(
        jax.ShapeDtypeStruct((NPAD, H1), jnp.float32),
        jax.ShapeDtypeStruct((NPAD, H1), jnp.float32),
        jax.ShapeDtypeStruct((N, 1), jnp.float32),
    ),
)


def _tc2h_body(pa_ref, y0h_ref, dinv_ref, xh_ref, bc0h_ref, g0h_ref,
               be0h_ref, x1h_ref):
  dinv = dinv_ref[...]
  agg = pa_ref[0, :N, :] + pa_ref[1, :N, :] + y0h_ref[:N, :]
  conv0 = dinv * agg + bc0h_ref[...][None, :]
  h = jnp.maximum(_bn(conv0, g0h_ref[...], be0h_ref[...]), 0.0)
  x1h_ref[...] = jnp.maximum(h + xh_ref[...], 0.0)


_tc2h = pl.pallas_call(
    _tc2h_body,
    out_shape=jax.ShapeDtypeStruct((N, H1), jnp.float32),
)


def _tc2c_body(x1a_ref, x1b_ref, dinv_ref, wc1_ref, ws1_ref, bs1_ref,
               y1_ref, r1_ref):
  dinv = dinv_ref[...]
  x1a = x1a_ref[...]
  x1b = x1b_ref[...]
  wc1 = wc1_ref[...]
  ws1 = ws1_ref[...]
  y1 = _matmul_t(x1a, wc1[:, :H1]) + _matmul_t(x1b, wc1[:, H1:])
  y1_ref[:N, :] = y1 * dinv
  y1_ref[N:, :] = jnp.zeros((NPAD - N, H1), jnp.float32)
  r1_ref[...] = (_matmul_t(x1a, ws1[:, :H1]) + _matmul_t(x1b, ws1[:, H1:])
                 + bs1_ref[...][None, :])


_tc2c = pl.pallas_call(
    _tc2c_body,
    out_shape=(
        jax.ShapeDtypeStruct((NPAD, H1), jnp.float32),
        jax.ShapeDtypeStruct((N, H1), jnp.float32),
    ),
)


def _tc3_body(aggp_ref, y1_ref, dinv_ref, r1_ref, batch_ref, bc1_ref, g1_ref,
              be1_ref, wd0_ref, bd0_ref, gd0_ref, bed0_ref, wd1_ref, bd1_ref,
              gd1_ref, bed1_ref, wd2_ref, bd2_ref, wp0_ref, bp0_ref, wp1_ref,
              bp1_ref, z_ref, xr_ref, zg_ref):
  dinv = dinv_ref[...]
  agg = aggp_ref[0, :N, :] + aggp_ref[1, :N, :] + y1_ref[:N, :]
  conv1 = dinv * agg + bc1_ref[...][None, :]
  h = jnp.maximum(_bn(conv1, g1_ref[...], be1_ref[...]), 0.0)
  x2 = jnp.maximum(h + r1_ref[...], 0.0)
  nrm = jnp.sqrt(jnp.sum(x2 * x2, axis=1, keepdims=True))
  z = x2 / jnp.maximum(nrm, 1e-12)
  z_ref[...] = z

  t = jnp.maximum(_matmul_t(z, wd0_ref[...]) + bd0_ref[...][None, :], 0.0)
  t = _bn(t, gd0_ref[...], bed0_ref[...])
  t = jnp.maximum(_matmul_t(t, wd1_ref[...]) + bd1_ref[...][None, :], 0.0)
  t = _bn(t, gd1_ref[...], bed1_ref[...])
  xr_ref[...] = _matmul_t(t, wd2_ref[...]) + bd2_ref[...][None, :]

  batch = batch_ref[...]
  rows_g = lax.broadcasted_iota(jnp.int32, (G, 1), 0)

  def seg_body(g, zg):
    m = (batch == g).astype(jnp.float32)
    row = jnp.max(z * m, axis=0)
    return jnp.where(rows_g == g, row[None, :], zg)

  zg = lax.fori_loop(0, G, seg_body, jnp.zeros((G, H1), jnp.float32))
  t = jnp.maximum(_matmul_t(zg, wp0_ref[...]) + bp0_ref[...][None, :], 0.0)
  zg_ref[...] = _matmul_t(t, wp1_ref[...]) + bp1_ref[...][None, :]


_tc3 = pl.pallas_call(
    _tc3_body,
    out_shape=(
        jax.ShapeDtypeStruct((N, H1), jnp.float32),
        jax.ShapeDtypeStruct((N, D), jnp.float32),
        jax.ShapeDtypeStruct((G, H1), jnp.float32),
    ),
)


def kernel(x, edge_index, batch, Wc0, bc0, g0, be0, Wc1, bc1, g1, be1, Ws1,
           bs1, Wd0, bd0, gd0, bed0, Wd1, bd1, gd1, bed1, Wd2, bd2, Wp0, bp0,
           Wp1, bp1):
  eidx = edge_index.reshape(2, NW, NCH, CH)
  degp = _deg_kernel(eidx)
  y0a, y0b, dinv = _tc1(x, Wc0, degp)
  p0a = _edge_agg(y0a, eidx)
  p0b = _edge_agg(y0b, eidx)
  x1a = _tc2h(p0a, y0a, dinv, x[:, :H1], bc0[:H1], g0[:H1], be0[:H1])
  x1b = _tc2h(p0b, y0b, dinv, x[:, H1:], bc0[H1:], g0[H1:], be0[H1:])
  y1, r1 = _tc2c(x1a, x1b, dinv, Wc1, Ws1, bs1)
  aggp1 = _edge_agg(y1, eidx)
  z, x_recon, z_g_mlp = _tc3(
      aggp1, y1, dinv, r1, batch.reshape(N, 1), bc1, g1, be1, Wd0, bd0, gd0,
      bed0, Wd1, bd1, gd1, bed1, Wd2, bd2, Wp0, bp0, Wp1, bp1
  )
  return (z, x_recon, z_g_mlp)


# speedup vs baseline: 20.2658x; 1.0383x over previous
import functools

import jax
import jax.numpy as jnp
from jax import lax
from jax.experimental import pallas as pl
from jax.experimental.pallas import tpu as pltpu
from jax.experimental.pallas import tpu_sc as plsc

N = 10000
E = 320000
D = 128
H0 = 128
H1 = 64
G = 64

NC = 2
NS = 16
NW = NC * NS
NPAD = 10240
NPS = NPAD // NS
EPW = E // NW
CH = 80
NCH = EPW // CH
NB = 5


def _zero_vmem(ref, rows, width):
  zv = jnp.zeros((16,), jnp.float32)

  def body(i, _):
    for j in range(width // 16):
      ref[i, pl.ds(j * 16, 16)] = zv
    return 0

  lax.fori_loop(0, rows, body, 0)


def _fill_vmem_1d(ref, rows, val):
  v = jnp.full((16,), val, jnp.float32)

  def body(i, _):
    ref[pl.ds(i * 16, 16)] = v
    return 0

  lax.fori_loop(0, rows // 16, body, 0)


_MESH = plsc.VectorSubcoreMesh(
    core_axis_name="c", subcore_axis_name="s", num_cores=NC, num_subcores=NS
)


@functools.partial(
    pl.kernel,
    out_type=jax.ShapeDtypeStruct((NC, NPAD), jnp.float32),
    mesh=_MESH,
    scratch_types=[
        pltpu.VMEM((NCH, CH), jnp.int32),
        pltpu.VMEM((CH,), jnp.float32),
        pltpu.VMEM_SHARED((NPAD,), jnp.float32),
        pltpu.SemaphoreType.DMA,
    ],
)
def _deg_kernel(eidx_hbm, out_hbm, didxv, ones, acc, sem):
  c = lax.axis_index("c")
  s = lax.axis_index("s")
  _fill_vmem_1d(ones, CH, 0.0)
  for k in range(NPS // CH):
    pltpu.sync_copy(ones, acc.at[pl.ds(s * NPS + k * CH, CH)])
  plsc.subcore_barrier()
  _fill_vmem_1d(ones, CH, 1.0)

  w = c * NS + s
  pltpu.sync_copy(eidx_hbm.at[1].at[w], didxv)

  def body(j, _):
    pltpu.sync_copy(ones, acc.at[didxv.at[j]], add=True)
    return 0

  lax.fori_loop(0, NCH, body, 0)
  plsc.subcore_barrier()
  pltpu.sync_copy(
      acc.at[pl.ds(s * NPS, NPS)], out_hbm.at[c].at[pl.ds(s * NPS, NPS)]
  )


F_AGG = 64


def _make_agg():
  F = F_AGG
  params = pltpu.CompilerParams(use_tc_tiling_on_sc=False)

  @functools.partial(
      pl.kernel,
      out_type=jax.ShapeDtypeStruct((NC, NPAD, F), jnp.float32),
      mesh=_MESH,
      compiler_params=params,
      scratch_types=[
          pltpu.VMEM((NCH, CH), jnp.int32),
          pltpu.VMEM((NCH, CH), jnp.int32),
          pltpu.VMEM((NB, CH, F), jnp.float32),
          pltpu.VMEM_SHARED((NPAD, F), jnp.float32),
          pltpu.SemaphoreType.DMA,
          pltpu.SemaphoreType.DMA,
          pltpu.SemaphoreType.DMA,
          pltpu.SemaphoreType.DMA,
          pltpu.SemaphoreType.DMA,
          pltpu.SemaphoreType.DMA,
          pltpu.SemaphoreType.DMA,
          pltpu.SemaphoreType.DMA,
          pltpu.SemaphoreType.DMA,
          pltpu.SemaphoreType.DMA,
      ],
      name=f"edge_agg_f{F}",
  )
  def agg(y_hbm, eidx_hbm, out_hbm, sidxv, didxv, rows, acc,
          sg0, sg1, sg2, sg3, sg4, ss0, ss1, ss2, ss3, ss4):
    semg = [sg0, sg1, sg2, sg3, sg4]
    sems = [ss0, ss1, ss2, ss3, ss4]
    c = lax.axis_index("c")
    s = lax.axis_index("s")
    _zero_vmem(rows.at[0], CH, F)
    for k in range(NPS // CH):
      pltpu.sync_copy(rows.at[0], acc.at[pl.ds(s * NPS + k * CH, CH)])
    plsc.subcore_barrier()

    w = c * NS + s
    pltpu.sync_copy(eidx_hbm.at[0].at[w], sidxv)
    pltpu.sync_copy(eidx_hbm.at[1].at[w], didxv)

    def body(t, _):
      gd = []
      for i in range(NB):
        j = t * NB + i
        gd.append(
            pltpu.async_copy(y_hbm.at[sidxv.at[j]], rows.at[i], semg[i])
        )
      sd = []
      for i in range(NB):
        j = t * NB + i
        gd[i].wait()
        sd.append(
            pltpu.async_copy(rows.at[i], acc.at[didxv.at[j]], sems[i],
                             add=True)
        )
      for i in range(NB):
        sd[i].wait()
      return 0

    lax.fori_loop(0, NCH // NB, body, 0)
    plsc.subcore_barrier()
    pltpu.sync_copy(
        acc.at[pl.ds(s * NPS, NPS)], out_hbm.at[c].at[pl.ds(s * NPS, NPS)]
    )

  return agg


_edge_agg = _make_agg()


def _matmul_t(a, w):
  return lax.dot_general(
      a, w, (((1,), (1,)), ((), ())), preferred_element_type=jnp.float32
  )


def _bn(t, g, b, eps=1e-5):
  m = jnp.mean(t, axis=0, keepdims=True)
  v = jnp.mean((t - m) * (t - m), axis=0, keepdims=True)
  return (t - m) * lax.rsqrt(v + eps) * g[None, :] + b[None, :]


def _tc1_body(x_ref, wc0_ref, degp_ref, y0a_ref, y0b_ref, dinv_ref):
  deg = degp_ref[0, :N] + degp_ref[1, :N] + 1.0
  dinv = lax.rsqrt(deg)[:, None]
  dinv_ref[...] = dinv
  y0 = _matmul_t(x_ref[...], wc0_ref[...]) * dinv
  zpad = jnp.zeros((NPAD - N, H1), jnp.float32)
  y0a_ref[:N, :] = y0[:, :H1]
  y0a_ref[N:, :] = zpad
  y0b_ref[:N, :] = y0[:, H1:]
  y0b_ref[N:, :] = zpad


_tc1 = pl.pallas_call(
    _tc1_body,
    out_shape=(
        jax.ShapeDtypeStruct((NPAD, H1), jnp.float32),
        jax.ShapeDtypeStruct((NPAD, H1), jnp.float32),
        jax.ShapeDtypeStruct((N, 1), jnp.float32),
    ),
)


def _tc2h_body(pa_ref, y0h_ref, dinv_ref, xh_ref, bc0h_ref, g0h_ref,
               be0h_ref, x1h_ref):
  dinv = dinv_ref[...]
  agg = pa_ref[0, :N, :] + pa_ref[1, :N, :] + y0h_ref[:N, :]
  conv0 = dinv * agg + bc0h_ref[...][None, :]
  h = jnp.maximum(_bn(conv0, g0h_ref[...], be0h_ref[...]), 0.0)
  x1h_ref[...] = jnp.maximum(h + xh_ref[...], 0.0)


_tc2h = pl.pallas_call(
    _tc2h_body,
    out_shape=jax.ShapeDtypeStruct((N, H1), jnp.float32),
)


def _tc2c_body(x1a_ref, x1b_ref, dinv_ref, wc1_ref, ws1_ref, bs1_ref,
               y1_ref, r1_ref):
  dinv = dinv_ref[...]
  x1a = x1a_ref[...]
  x1b = x1b_ref[...]
  wc1 = wc1_ref[...]
  ws1 = ws1_ref[...]
  y1 = _matmul_t(x1a, wc1[:, :H1]) + _matmul_t(x1b, wc1[:, H1:])
  y1_ref[:N, :] = y1 * dinv
  y1_ref[N:, :] = jnp.zeros((NPAD - N, H1), jnp.float32)
  r1_ref[...] = (_matmul_t(x1a, ws1[:, :H1]) + _matmul_t(x1b, ws1[:, H1:])
                 + bs1_ref[...][None, :])


_tc2c = pl.pallas_call(
    _tc2c_body,
    out_shape=(
        jax.ShapeDtypeStruct((NPAD, H1), jnp.float32),
        jax.ShapeDtypeStruct((N, H1), jnp.float32),
    ),
)


def _tc3_body(aggp_ref, y1_ref, dinv_ref, r1_ref, batch_ref, bc1_ref, g1_ref,
              be1_ref, wd0_ref, bd0_ref, gd0_ref, bed0_ref, wd1_ref, bd1_ref,
              gd1_ref, bed1_ref, wd2_ref, bd2_ref, wp0_ref, bp0_ref, wp1_ref,
              bp1_ref, z_ref, xr_ref, zg_ref):
  dinv = dinv_ref[...]
  agg = aggp_ref[0, :N, :] + aggp_ref[1, :N, :] + y1_ref[:N, :]
  conv1 = dinv * agg + bc1_ref[...][None, :]
  h = jnp.maximum(_bn(conv1, g1_ref[...], be1_ref[...]), 0.0)
  x2 = jnp.maximum(h + r1_ref[...], 0.0)
  nrm = jnp.sqrt(jnp.sum(x2 * x2, axis=1, keepdims=True))
  z = x2 / jnp.maximum(nrm, 1e-12)
  z_ref[...] = z

  t = jnp.maximum(_matmul_t(z, wd0_ref[...]) + bd0_ref[...][None, :], 0.0)
  t = _bn(t, gd0_ref[...], bed0_ref[...])
  t = jnp.maximum(_matmul_t(t, wd1_ref[...]) + bd1_ref[...][None, :], 0.0)
  t = _bn(t, gd1_ref[...], bed1_ref[...])
  xr_ref[...] = _matmul_t(t, wd2_ref[...]) + bd2_ref[...][None, :]

  batch = batch_ref[...]
  rows_g = lax.broadcasted_iota(jnp.int32, (G, 1), 0)

  def seg_body(g, zg):
    m = (batch == g).astype(jnp.float32)
    row = jnp.max(z * m, axis=0)
    return jnp.where(rows_g == g, row[None, :], zg)

  zg = lax.fori_loop(0, G, seg_body, jnp.zeros((G, H1), jnp.float32))
  t = jnp.maximum(_matmul_t(zg, wp0_ref[...]) + bp0_ref[...][None, :], 0.0)
  zg_ref[...] = _matmul_t(t, wp1_ref[...]) + bp1_ref[...][None, :]


_tc3 = pl.pallas_call(
    _tc3_body,
    out_shape=(
        jax.ShapeDtypeStruct((N, H1), jnp.float32),
        jax.ShapeDtypeStruct((N, D), jnp.float32),
        jax.ShapeDtypeStruct((G, H1), jnp.float32),
    ),
)


def kernel(x, edge_index, batch, Wc0, bc0, g0, be0, Wc1, bc1, g1, be1, Ws1,
           bs1, Wd0, bd0, gd0, bed0, Wd1, bd1, gd1, bed1, Wd2, bd2, Wp0, bp0,
           Wp1, bp1):
  eidx = edge_index.reshape(2, NW, NCH, CH)
  degp = _deg_kernel(eidx)
  y0a, y0b, dinv = _tc1(x, Wc0, degp)
  p0a = _edge_agg(y0a, eidx)
  p0b = _edge_agg(y0b, eidx)
  x1a = _tc2h(p0a, y0a, dinv, x[:, :H1], bc0[:H1], g0[:H1], be0[:H1])
  x1b = _tc2h(p0b, y0b, dinv, x[:, H1:], bc0[H1:], g0[H1:], be0[H1:])
  y1, r1 = _tc2c(x1a, x1b, dinv, Wc1, Ws1, bs1)
  aggp1 = _edge_agg(y1, eidx)
  z, x_recon, z_g_mlp = _tc3(
      aggp1, y1, dinv, r1, batch.reshape(N, 1), bc1, g1, be1, Wd0, bd0, gd0,
      bed0, Wd1, bd1, gd1, bed1, Wd2, bd2, Wp0, bp0, Wp1, bp1
  )
  return (z, x_recon, z_g_mlp)
